# Initial kernel scaffold; baseline (speedup 1.0000x reference)
import jax
import jax.numpy as jnp
from jax.experimental import pallas as pl


def kernel(x, emb_weight):
    raise NotImplementedError("write your pallas kernel here")


# speedup vs baseline: 1.1135x; 1.1135x over previous
import jax
import jax.numpy as jnp
from jax import lax
from jax.experimental import pallas as pl
from jax.experimental.pallas import tpu as pltpu
from jax.experimental.pallas import tpu_sc as plsc

NUM_TOKENS = 1000000
DIM = 32
BATCH = 16384
HIST = 50

_N = BATCH * HIST
_NW = 32
_PER_W = _N // _NW
_C = 1280
_NSTEPS = _PER_W // _C


def _emb_body(table, xflat, out, idx_v, rows0, rows1, gs0, gs1, os0, os1):
    c = lax.axis_index("c")
    s = lax.axis_index("s")
    wid = s * 2 + c
    base = wid * _PER_W
    pltpu.sync_copy(xflat.at[pl.ds(base, _PER_W)], idx_v)

    rows = (rows0, rows1)
    gs = (gs0, gs1)
    os_ = (os0, os1)
    gh = [None, None]
    oh = [None, None]

    def start(step, b):
        off = step * _C
        gh[b] = pltpu.async_copy(table.at[idx_v.at[pl.ds(off, _C)]],
                                 rows[b], gs[b])

    start(0, 0)
    for i in range(_NSTEPS):
        b = i % 2
        nb = 1 - b
        if i >= 1:
            oh[nb].wait()
        if i + 1 < _NSTEPS:
            start(i + 1, nb)
        gh[b].wait()
        oh[b] = pltpu.async_copy(rows[b], out.at[pl.ds(base + i * _C, _C)],
                                 os_[b])
    oh[(_NSTEPS - 1) % 2].wait()


@jax.jit
def _emb(xflat, table):
    mesh = plsc.VectorSubcoreMesh(core_axis_name="c", subcore_axis_name="s")
    f = pl.kernel(
        _emb_body,
        mesh=mesh,
        compiler_params=pltpu.CompilerParams(use_tc_tiling_on_sc=False),
        out_type=jax.ShapeDtypeStruct((_N, DIM), jnp.float32),
        scratch_types=[
            pltpu.VMEM((_PER_W,), jnp.int32),
            pltpu.VMEM((_C, DIM), jnp.float32),
            pltpu.VMEM((_C, DIM), jnp.float32),
            pltpu.SemaphoreType.DMA,
            pltpu.SemaphoreType.DMA,
            pltpu.SemaphoreType.DMA,
            pltpu.SemaphoreType.DMA,
        ],
    )
    return f(table, xflat)


def kernel(x, emb_weight):
    xflat = x.astype(jnp.int32).reshape(_N)
    out = _emb(xflat, emb_weight)
    return out.reshape(BATCH, HIST, DIM)
